# baseline (device time: 60905 ns/iter reference)
import jax
import jax.numpy as jnp
from jax import lax
from jax.experimental import pallas as pl
from jax.experimental.pallas import tpu as pltpu

B = 16
NB = 128
BS = 16
H = 16
D = 64
P_LOCAL = 128
T_LOCAL = P_LOCAL * BS
SCALE = D ** -0.5
NEG = -1e30


def kernel(Q, K, V, bt, lens):
    q = (Q.reshape(B, H, D) * SCALE).astype(jnp.bfloat16).swapaxes(0, 1)
    k = K.reshape(T_LOCAL, H, D)
    v = V.reshape(T_LOCAL, H, D)
    lens2 = lens.reshape(B, 1)

    def body(q_ref, k_ref, v_ref, bt_ref, lens_ref, out_ref,
             logw_ref, k_buf, v_buf, o_send, st_send, o_recv, st_recv,
             k_sems, v_sems, send_sems, recv_sems):
        h = pl.program_id(0)
        my_x = lax.axis_index("x")
        my_y = lax.axis_index("y")
        peer = (1 - my_x, my_y)

        def fetch(head, slot):
            pltpu.make_async_copy(
                k_ref.at[:, head, :], k_buf.at[slot], k_sems.at[slot]
            ).start()
            pltpu.make_async_copy(
                v_ref.at[:, head, :], v_buf.at[slot], v_sems.at[slot]
            ).start()

        @pl.when(h == 0)
        def _prologue():
            fetch(0, 0)
            fetch(1, 1)

            barrier = pltpu.get_barrier_semaphore()
            pl.semaphore_signal(barrier, inc=1, device_id=peer,
                                device_id_type=pl.DeviceIdType.MESH)
            pl.semaphore_wait(barrier, 1)

            x_off = my_x * P_LOCAL
            bt_arr = bt_ref[...]
            lens_arr = lens_ref[...]
            slot = lax.broadcasted_iota(jnp.int32, (B, NB, P_LOCAL), 1)
            page = lax.broadcasted_iota(jnp.int32, (B, NB, P_LOCAL), 2)
            hit = (bt_arr[:, :, None] == page + x_off) & (
                slot < lens_arr[:, :, None])
            w = jnp.sum(hit.astype(jnp.float32), axis=1)

            tpage = lax.broadcasted_iota(
                jnp.int32, (P_LOCAL, T_LOCAL), 1) // BS
            prow = lax.broadcasted_iota(jnp.int32, (P_LOCAL, T_LOCAL), 0)
            expand = (tpage == prow).astype(jnp.bfloat16)
            w_tok = lax.dot_general(
                w.astype(jnp.bfloat16), expand,
                (((1,), (0,)), ((), ())),
                preferred_element_type=jnp.float32,
            )
            logw_ref[...] = jnp.where(w_tok > 0, jnp.log(w_tok), NEG)

        slot = lax.rem(h, 3)
        pltpu.make_async_copy(
            k_ref.at[:, h, :], k_buf.at[slot], k_sems.at[slot]
        ).wait()
        pltpu.make_async_copy(
            v_ref.at[:, h, :], v_buf.at[slot], v_sems.at[slot]
        ).wait()

        @pl.when(h + 2 < H)
        def _prefetch():
            fetch(h + 2, lax.rem(h + 2, 3))

        qh = q_ref[h]
        kh = k_buf[slot].astype(jnp.bfloat16)
        s = lax.dot_general(
            qh, kh, (((1,), (1,)), ((), ())),
            preferred_element_type=jnp.float32,
        ) + logw_ref[...]
        m_h = jnp.max(s, axis=1, keepdims=True)
        p_un = jnp.exp((s - m_h).astype(jnp.bfloat16))
        l_h = jnp.sum(p_un, axis=1, keepdims=True,
                      dtype=jnp.float32)
        vh = v_buf[slot].astype(jnp.bfloat16)
        o_h = lax.dot_general(
            p_un, vh, (((1,), (0,)), ((), ())),
            preferred_element_type=jnp.float32,
        )

        o_send[pl.ds(h, 1)] = o_h[None, :, :]
        st_send[pl.ds(h, 1)] = jnp.concatenate([m_h, l_h], axis=1)[None]

        @pl.when(h == H - 1)
        def _epilogue():
            rdma_o = pltpu.make_async_remote_copy(
                src_ref=o_send, dst_ref=o_recv,
                send_sem=send_sems.at[0], recv_sem=recv_sems.at[0],
                device_id=peer, device_id_type=pl.DeviceIdType.MESH,
            )
            rdma_st = pltpu.make_async_remote_copy(
                src_ref=st_send, dst_ref=st_recv,
                send_sem=send_sems.at[1], recv_sem=recv_sems.at[1],
                device_id=peer, device_id_type=pl.DeviceIdType.MESH,
            )
            rdma_o.start()
            rdma_st.start()
            rdma_o.wait()
            rdma_st.wait()

            m_loc = st_send[:, :, 0:1]
            l_loc = st_send[:, :, 1:2]
            o_loc = o_send[...]
            m_p = st_recv[:, :, 0:1]
            l_p = st_recv[:, :, 1:2]
            o_p = o_recv[...]
            m_new = jnp.maximum(m_loc, m_p)
            a = jnp.exp(m_loc - m_new)
            c = jnp.exp(m_p - m_new)
            l_new = l_loc * a + l_p * c
            out_ref[...] = (o_loc * a + o_p * c) / l_new

    out = pl.pallas_call(
        body,
        grid=(H,),
        out_shape=jax.ShapeDtypeStruct((H, B, D), jnp.float32),
        in_specs=[
            pl.BlockSpec(memory_space=pltpu.VMEM),
            pl.BlockSpec(memory_space=pl.ANY),
            pl.BlockSpec(memory_space=pl.ANY),
            pl.BlockSpec(memory_space=pltpu.VMEM),
            pl.BlockSpec(memory_space=pltpu.VMEM),
        ],
        out_specs=pl.BlockSpec((H, B, D), lambda h: (0, 0, 0)),
        scratch_shapes=[
            pltpu.VMEM((B, T_LOCAL), jnp.float32),
            pltpu.VMEM((3, T_LOCAL, D), jnp.float32),
            pltpu.VMEM((3, T_LOCAL, D), jnp.float32),
            pltpu.VMEM((H, B, D), jnp.float32),
            pltpu.VMEM((H, B, 2), jnp.float32),
            pltpu.VMEM((H, B, D), jnp.float32),
            pltpu.VMEM((H, B, 2), jnp.float32),
            pltpu.SemaphoreType.DMA((3,)),
            pltpu.SemaphoreType.DMA((3,)),
            pltpu.SemaphoreType.DMA((2,)),
            pltpu.SemaphoreType.DMA((2,)),
        ],
        compiler_params=pltpu.CompilerParams(collective_id=0),
    )(q, k, v, bt, lens2)

    return out.swapaxes(0, 1).reshape(B, 1, H, D)
